# baseline (speedup 1.0000x reference)
import functools

import jax
import jax.numpy as jnp
from jax import lax
from jax.experimental import pallas as pl
from jax.experimental.pallas import tpu as pltpu
from jax.experimental.pallas import tpu_sc as plsc

N = 10000
E = 320000
BD = 128
H = 128
HALF = H // 2
NUM_LAYERS = 3

NC = 2
NS = 16
CHUNK = 128
EPT = 20480
EP = EPT * NS
CPT = EPT // CHUNK
SUP = 16
LSUP = CPT // SUP
RPT = N // NS
ZB = 125
LANES = 16
NBUF = 4

_SPLAT_DN = lax.GatherDimensionNumbers(
    offset_dims=(), collapsed_slice_dims=(0,), start_index_map=(0,))


def _splat(vec, e):
    idx = jnp.full((LANES, 1), e, jnp.int32)
    return lax.gather(vec, idx, _SPLAT_DN, (1,),
                      mode=lax.GatherScatterMode.PROMISE_IN_BOUNDS)


def _encoder_body(x_ref, W1_ref, b1_ref, W2_ref, b2_ref, Wzk_ref, bzk_ref,
                  hs_ref, zk_ref):
    h1 = jnp.maximum(
        jnp.dot(x_ref[...], W1_ref[...], preferred_element_type=jnp.float32)
        + b1_ref[...], 0.0)
    h = (jnp.dot(h1, W2_ref[...], preferred_element_type=jnp.float32)
         + b2_ref[...])
    zk_ref[...] = (jnp.dot(h, Wzk_ref[...], preferred_element_type=jnp.float32)
                   + bzk_ref[...])
    hs_ref[0] = h[:, :HALF]
    hs_ref[1] = h[:, HALF:]


def _encoder(x, W1, b1, W2, b2, Wzk, bzk):
    BR = 1000
    return pl.pallas_call(
        _encoder_body,
        grid=(N // BR,),
        in_specs=[
            pl.BlockSpec((BR, BD), lambda i: (i, 0)),
            pl.BlockSpec((BD, H), lambda i: (0, 0)),
            pl.BlockSpec((1, H), lambda i: (0, 0)),
            pl.BlockSpec((H, H), lambda i: (0, 0)),
            pl.BlockSpec((1, H), lambda i: (0, 0)),
            pl.BlockSpec((H, 64), lambda i: (0, 0)),
            pl.BlockSpec((1, 64), lambda i: (0, 0)),
        ],
        out_specs=[
            pl.BlockSpec((2, BR, HALF), lambda i: (0, i, 0)),
            pl.BlockSpec((BR, 64), lambda i: (i, 0)),
        ],
        out_shape=[
            jax.ShapeDtypeStruct((2, N, HALF), jnp.float32),
            jax.ShapeDtypeStruct((N, 64), jnp.float32),
        ],
    )(x, W1, b1, W2, b2, Wzk, bzk)


_mesh = plsc.VectorSubcoreMesh(core_axis_name="c", subcore_axis_name="s",
                               num_cores=NC, num_subcores=NS)


@functools.partial(
    pl.kernel,
    out_type=jax.ShapeDtypeStruct((2 * N, HALF), jnp.float32),
    mesh=_mesh,
    compiler_params=pltpu.CompilerParams(use_tc_tiling_on_sc=False,
                                         needs_layout_passes=False),
    scratch_types=[
        pltpu.VMEM((2, SUP, CHUNK), jnp.int32),
        pltpu.VMEM((2, SUP, CHUNK), jnp.float32),
        pltpu.VMEM((2, CHUNK), jnp.int32),
        pltpu.VMEM((2, CHUNK), jnp.int32),
        pltpu.VMEM((NBUF, CHUNK, HALF), jnp.float32),
        pltpu.VMEM((ZB, HALF), jnp.float32),
        pltpu.VMEM_SHARED((N, HALF), jnp.float32),
        pltpu.VMEM_SHARED((N, HALF), jnp.float32),
        pltpu.SemaphoreType.DMA,
        pltpu.SemaphoreType.DMA,
        pltpu.SemaphoreType.DMA,
        pltpu.SemaphoreType.DMA,
        pltpu.SemaphoreType.DMA,
        pltpu.SemaphoreType.DMA,
    ],
)
def _mp_kernel(h0, packp, attrp, out,
               packS, attrS, gidx, dstb, rows, zbuf, stateA, stateB,
               sg0, sg1, ss0, ss1, sl0, sl1):
    c = lax.axis_index("c")
    s = lax.axis_index("s")
    row0 = s * RPT
    ebase = s * CPT
    sg = (sg0, sg1)
    ss = (ss0, ss1)

    def unpack_gidx(cc, slot, hbm=False):
        sup_slot = lax.shift_right_logical(cc, 4) & 1
        ic = cc & 15
        base = c * N if hbm else 0
        for v in range(CHUNK // LANES):
            vsl = pl.ds(v * LANES, LANES)
            w = packS[sup_slot, ic, vsl]
            gidx[slot, vsl] = (w & 0xFFFF) + base

    def unpack_dst(cc, slot):
        sup_slot = lax.shift_right_logical(cc, 4) & 1
        ic = cc & 15
        for v in range(CHUNK // LANES):
            vsl = pl.ds(v * LANES, LANES)
            w = packS[sup_slot, ic, vsl]
            dstb[slot, vsl] = lax.shift_right_logical(w, 16)

    def load_super(u, slot):
        off = ebase + u * SUP
        pltpu.async_copy(packp.at[pl.ds(off, SUP)], packS.at[slot], sl0)
        pltpu.async_copy(attrp.at[pl.ds(off, SUP)], attrS.at[slot], sl0)

    def wait_super(slot):
        pltpu.make_async_copy(packp.at[pl.ds(0, SUP)], packS.at[slot],
                              sl0).wait()
        pltpu.make_async_copy(attrp.at[pl.ds(0, SUP)], attrS.at[slot],
                              sl0).wait()

    pltpu.sync_copy(h0.at[pl.ds(c * N + row0, RPT)],
                    stateA.at[pl.ds(row0, RPT)])

    def zb_body(i, carry):
        for cs in range(HALF // LANES):
            zbuf[i, pl.ds(cs * LANES, LANES)] = jnp.zeros((LANES,),
                                                          jnp.float32)
        return carry

    lax.fori_loop(0, ZB, zb_body, 0)
    for p in range(RPT // ZB):
        pltpu.sync_copy(zbuf, stateB.at[pl.ds(row0 + p * ZB, ZB)])
    plsc.subcore_barrier()

    for layer in range(NUM_LAYERS):
        read = stateA if layer % 2 == 0 else stateB
        write = stateB if layer % 2 == 0 else stateA
        hread = h0 if layer == 0 else out
        apply_relu = layer > 0

        load_super(0, 0)
        wait_super(0)
        load_super(1, 1)
        unpack_gidx(0, 0)
        pltpu.async_copy(read.at[gidx.at[0]], rows.at[0], sg0)
        unpack_gidx(1, 1)
        pltpu.async_copy(read.at[gidx.at[1]], rows.at[1], sg1)

        def ring_body(it4, carry):
            for b in range(NBUF):
                cc = it4 * NBUF + b
                pb = b % 2
                semg = sg[pb]
                sems = ss[pb]
                pltpu.make_async_copy(read.at[gidx.at[pb]],
                                      rows.at[b], semg).wait()
                @pl.when(cc >= 2)
                def _():
                    pltpu.make_async_copy(
                        rows.at[(b + 2) % NBUF],
                        write.at[dstb.at[pb]], sems).wait()

                if b == 0:
                    nu0 = lax.shift_right_logical(cc, 4) + 1

                    @pl.when(jnp.logical_and((cc & 15) == 0,
                                             jnp.logical_and(
                                                 cc >= SUP,
                                                 nu0 <= LSUP - 1)))
                    def _():
                        load_super(nu0, nu0 & 1)
                if b == 2:
                    nu2 = lax.shift_right_logical(cc, 4) + 1

                    @pl.when(jnp.logical_and((cc & 15) == SUP - 2,
                                             nu2 <= LSUP - 1))
                    def _():
                        wait_super(nu2 & 1)

                hbm_gather = (b + 2) % NBUF == 3

                @pl.when(cc + 2 < CPT)
                def _():
                    unpack_gidx(cc + 2, pb, hbm=hbm_gather)
                    gsrc = hread if hbm_gather else read
                    pltpu.async_copy(gsrc.at[gidx.at[pb]],
                                     rows.at[(b + 2) % NBUF], semg)

                unpack_dst(cc, pb)

                sup_slot = lax.shift_right_logical(cc, 4) & 1
                ic = cc & 15

                @plsc.parallel_loop(0, CHUNK // LANES)
                def group_body(k):
                    av = attrS[sup_slot, ic, pl.ds(k * LANES, LANES)]
                    for e in range(LANES):
                        ae = _splat(av, e)
                        j = k * LANES + e
                        for cs in range(HALF // LANES):
                            vsl = pl.ds(cs * LANES, LANES)
                            r = rows[b, j, vsl]
                            if apply_relu:
                                r = jnp.maximum(r, 0.0)
                            rows[b, j, vsl] = r * ae
                pltpu.async_copy(rows.at[b], write.at[dstb.at[pb]], sems,
                                 add=True)
            return carry

        lax.fori_loop(0, CPT // NBUF, ring_body, 0)
        pltpu.make_async_copy(rows.at[(CPT - 2) % NBUF],
                              write.at[dstb.at[0]], ss[0]).wait()
        pltpu.make_async_copy(rows.at[(CPT - 1) % NBUF],
                              write.at[dstb.at[1]], ss[1]).wait()
        plsc.subcore_barrier()

        pltpu.sync_copy(write.at[pl.ds(row0, RPT)],
                        out.at[pl.ds(c * N + row0, RPT)])
        if layer < NUM_LAYERS - 1:
            for p in range(RPT // ZB):
                pltpu.sync_copy(zbuf, read.at[pl.ds(row0 + p * ZB, ZB)])
        plsc.subcore_barrier()


def _head_body(F_ref, Wphi_ref, bphi_ref, Wt_ref, bt_ref,
               gs_ref, phi_ref, ts_ref):
    s0 = jnp.sum(jnp.maximum(F_ref[0], 0.0), axis=0, keepdims=True)
    s1 = jnp.sum(jnp.maximum(F_ref[1], 0.0), axis=0, keepdims=True)
    gs = jnp.concatenate([s0, s1], axis=1) * (1.0 / N)
    gs_ref[...] = gs
    phi_ref[...] = (jnp.dot(gs, Wphi_ref[...],
                            preferred_element_type=jnp.float32)
                    + bphi_ref[...])
    ts_ref[...] = (jnp.dot(gs, Wt_ref[...],
                           preferred_element_type=jnp.float32)
                   + bt_ref[...])


def _head(F, Wphi, bphi, Wt, bt):
    return pl.pallas_call(
        _head_body,
        out_shape=[
            jax.ShapeDtypeStruct((1, H), jnp.float32),
            jax.ShapeDtypeStruct((1, 1), jnp.float32),
            jax.ShapeDtypeStruct((1, 256), jnp.float32),
        ],
    )(F, Wphi, bphi, Wt, bt)


def kernel(x, edge_index, edge_attr, W1, b1, W2, b2, Wzk, bzk, Wphi, bphi,
           Wt, bt):
    hs, zk = _encoder(x, W1, b1.reshape(1, H), W2, b2.reshape(1, H),
                      Wzk, bzk.reshape(1, 64))
    h0 = hs.reshape(2 * N, HALF)

    pad = EP - E
    packed = jnp.concatenate(
        [edge_index[0] | (edge_index[1] << 16),
         jnp.zeros((pad,), jnp.int32)]).reshape(NS * CPT, CHUNK)
    attr = jnp.concatenate(
        [edge_attr[:, 0], jnp.zeros((pad,), jnp.float32)]).reshape(
            NS * CPT, CHUNK)

    s3 = _mp_kernel(h0, packed, attr)
    F = s3.reshape(2, N, HALF)

    gs, phi_q, temporal_sig = _head(F, Wphi, bphi.reshape(1, 1), Wt,
                                    bt.reshape(1, 256))
    return (phi_q, temporal_sig, zk, gs)


# speedup vs baseline: 1.3093x; 1.3093x over previous
import functools

import jax
import jax.numpy as jnp
from jax import lax
from jax.experimental import pallas as pl
from jax.experimental.pallas import tpu as pltpu
from jax.experimental.pallas import tpu_sc as plsc

N = 10000
E = 320000
BD = 128
H = 128
HALF = H // 2
NUM_LAYERS = 3

NC = 2
NS = 16
CHUNK = 128
EPT = 20480
EP = EPT * NS
CPT = EPT // CHUNK
SUP = 16
LSUP = CPT // SUP
RPT = N // NS
ZB = 125
LANES = 16
NBUF = 4

_SPLAT_DN = lax.GatherDimensionNumbers(
    offset_dims=(), collapsed_slice_dims=(0,), start_index_map=(0,))


def _splat(vec, e):
    idx = jnp.full((LANES, 1), e, jnp.int32)
    return lax.gather(vec, idx, _SPLAT_DN, (1,),
                      mode=lax.GatherScatterMode.PROMISE_IN_BOUNDS)


def _encoder_body(x_ref, W1_ref, b1_ref, W2_ref, b2_ref, Wzk_ref, bzk_ref,
                  hs_ref, zk_ref):
    h1 = jnp.maximum(
        jnp.dot(x_ref[...], W1_ref[...], preferred_element_type=jnp.float32)
        + b1_ref[...], 0.0)
    h = (jnp.dot(h1, W2_ref[...], preferred_element_type=jnp.float32)
         + b2_ref[...])
    zk_ref[...] = (jnp.dot(h, Wzk_ref[...], preferred_element_type=jnp.float32)
                   + bzk_ref[...])
    hs_ref[0] = h[:, :HALF]
    hs_ref[1] = h[:, HALF:]


def _encoder(x, W1, b1, W2, b2, Wzk, bzk):
    BR = 1000
    return pl.pallas_call(
        _encoder_body,
        grid=(N // BR,),
        in_specs=[
            pl.BlockSpec((BR, BD), lambda i: (i, 0)),
            pl.BlockSpec((BD, H), lambda i: (0, 0)),
            pl.BlockSpec((1, H), lambda i: (0, 0)),
            pl.BlockSpec((H, H), lambda i: (0, 0)),
            pl.BlockSpec((1, H), lambda i: (0, 0)),
            pl.BlockSpec((H, 64), lambda i: (0, 0)),
            pl.BlockSpec((1, 64), lambda i: (0, 0)),
        ],
        out_specs=[
            pl.BlockSpec((2, BR, HALF), lambda i: (0, i, 0)),
            pl.BlockSpec((BR, 64), lambda i: (i, 0)),
        ],
        out_shape=[
            jax.ShapeDtypeStruct((2, N, HALF), jnp.float32),
            jax.ShapeDtypeStruct((N, 64), jnp.float32),
        ],
    )(x, W1, b1, W2, b2, Wzk, bzk)


_mesh = plsc.VectorSubcoreMesh(core_axis_name="c", subcore_axis_name="s",
                               num_cores=NC, num_subcores=NS)


@functools.partial(
    pl.kernel,
    out_type=jax.ShapeDtypeStruct((2 * N, HALF), jnp.float32),
    mesh=_mesh,
    compiler_params=pltpu.CompilerParams(use_tc_tiling_on_sc=False,
                                         needs_layout_passes=False),
    scratch_types=[
        pltpu.VMEM((2, SUP, CHUNK), jnp.int32),
        pltpu.VMEM((2, SUP, CHUNK), jnp.float32),
        pltpu.VMEM((2, CHUNK), jnp.int32),
        pltpu.VMEM((2, 2, CHUNK // 2), jnp.int32),
        pltpu.VMEM((NBUF, CHUNK, HALF), jnp.float32),
        pltpu.VMEM((ZB, HALF), jnp.float32),
        pltpu.VMEM_SHARED((N, HALF), jnp.float32),
        pltpu.VMEM_SHARED((N, HALF), jnp.float32),
        pltpu.SemaphoreType.DMA,
        pltpu.SemaphoreType.DMA,
        pltpu.SemaphoreType.DMA,
        pltpu.SemaphoreType.DMA,
        pltpu.SemaphoreType.DMA,
        pltpu.SemaphoreType.DMA,
    ],
)
def _mp_kernel(h0, packp, attrp, out,
               packS, attrS, gidx, dstb, rows, zbuf, stateA, stateB,
               sg0, sg1, ss0, ss1, sl0, sl1):
    c = lax.axis_index("c")
    s = lax.axis_index("s")
    row0 = s * RPT
    ebase = s * CPT
    sg = (sg0, sg1)
    ss = (ss0, ss1)

    def unpack_gidx(cc, slot):
        sup_slot = lax.shift_right_logical(cc, 4) & 1
        ic = cc & 15
        for v in range(CHUNK // LANES):
            vsl = pl.ds(v * LANES, LANES)
            w = packS[sup_slot, ic, vsl]
            gidx[slot, vsl] = w & 0xFFFF

    def unpack_dst(cc, slot):
        sup_slot = lax.shift_right_logical(cc, 4) & 1
        ic = cc & 15
        for v in range(CHUNK // LANES):
            vsl = pl.ds(v * LANES, LANES)
            w = packS[sup_slot, ic, vsl]
            dstb[slot, v // 4, pl.ds((v % 4) * LANES, LANES)] = (
                lax.shift_right_logical(w, 16))

    def load_super(u, slot):
        off = ebase + u * SUP
        pltpu.async_copy(packp.at[pl.ds(off, SUP)], packS.at[slot], sl0)
        pltpu.async_copy(attrp.at[pl.ds(off, SUP)], attrS.at[slot], sl0)

    def wait_super(slot):
        pltpu.make_async_copy(packp.at[pl.ds(0, SUP)], packS.at[slot],
                              sl0).wait()
        pltpu.make_async_copy(attrp.at[pl.ds(0, SUP)], attrS.at[slot],
                              sl0).wait()

    pltpu.sync_copy(h0.at[pl.ds(c * N + row0, RPT)],
                    stateA.at[pl.ds(row0, RPT)])

    def zb_body(i, carry):
        for cs in range(HALF // LANES):
            zbuf[i, pl.ds(cs * LANES, LANES)] = jnp.zeros((LANES,),
                                                          jnp.float32)
        return carry

    lax.fori_loop(0, ZB, zb_body, 0)
    for p in range(RPT // ZB):
        pltpu.sync_copy(zbuf, stateB.at[pl.ds(row0 + p * ZB, ZB)])
    plsc.subcore_barrier()

    for layer in range(NUM_LAYERS):
        read = stateA if layer % 2 == 0 else stateB
        write = stateB if layer % 2 == 0 else stateA
        apply_relu = layer > 0

        load_super(0, 0)
        wait_super(0)
        load_super(1, 1)
        unpack_gidx(0, 0)
        pltpu.async_copy(read.at[gidx.at[0]], rows.at[0], sg0)
        unpack_gidx(1, 1)
        pltpu.async_copy(read.at[gidx.at[1]], rows.at[1], sg1)

        def ring_body(it4, carry):
            for b in range(NBUF):
                cc = it4 * NBUF + b
                pb = b % 2
                semg = sg[pb]
                sems = ss[pb]
                pltpu.make_async_copy(read.at[gidx.at[pb]],
                                      rows.at[b], semg).wait()
                @pl.when(cc >= 2)
                def _():
                    for half in range(2):
                        hr = pl.ds(half * (CHUNK // 2), CHUNK // 2)
                        pltpu.make_async_copy(
                            rows.at[(b + 2) % NBUF].at[hr],
                            write.at[dstb.at[pb, half]], sems).wait()

                if b == 0:
                    nu0 = lax.shift_right_logical(cc, 4) + 1

                    @pl.when(jnp.logical_and((cc & 15) == 0,
                                             jnp.logical_and(
                                                 cc >= SUP,
                                                 nu0 <= LSUP - 1)))
                    def _():
                        load_super(nu0, nu0 & 1)
                if b == 2:
                    nu2 = lax.shift_right_logical(cc, 4) + 1

                    @pl.when(jnp.logical_and((cc & 15) == SUP - 2,
                                             nu2 <= LSUP - 1))
                    def _():
                        wait_super(nu2 & 1)

                @pl.when(cc + 2 < CPT)
                def _():
                    unpack_gidx(cc + 2, pb)
                    pltpu.async_copy(read.at[gidx.at[pb]],
                                     rows.at[(b + 2) % NBUF], semg)

                unpack_dst(cc, pb)

                sup_slot = lax.shift_right_logical(cc, 4) & 1
                ic = cc & 15

                for half in range(2):
                    h_lo = half * (CHUNK // LANES // 2)
                    h_hi = (half + 1) * (CHUNK // LANES // 2)

                    @plsc.parallel_loop(h_lo, h_hi)
                    def group_body(k):
                        av = attrS[sup_slot, ic, pl.ds(k * LANES, LANES)]
                        for e in range(LANES):
                            ae = _splat(av, e)
                            j = k * LANES + e
                            for cs in range(HALF // LANES):
                                vsl = pl.ds(cs * LANES, LANES)
                                r = rows[b, j, vsl]
                                if apply_relu:
                                    r = jnp.maximum(r, 0.0)
                                rows[b, j, vsl] = r * ae

                    hr = pl.ds(half * (CHUNK // 2), CHUNK // 2)
                    pltpu.async_copy(rows.at[b].at[hr],
                                     write.at[dstb.at[pb, half]], sems,
                                     add=True)
            return carry

        lax.fori_loop(0, CPT // NBUF, ring_body, 0)
        for half in range(2):
            hr = pl.ds(half * (CHUNK // 2), CHUNK // 2)
            pltpu.make_async_copy(rows.at[(CPT - 2) % NBUF].at[hr],
                                  write.at[dstb.at[0, half]], ss[0]).wait()
            pltpu.make_async_copy(rows.at[(CPT - 1) % NBUF].at[hr],
                                  write.at[dstb.at[1, half]], ss[1]).wait()
        plsc.subcore_barrier()

        if layer < NUM_LAYERS - 1:
            for p in range(RPT // ZB):
                pltpu.sync_copy(zbuf, read.at[pl.ds(row0 + p * ZB, ZB)])
        else:
            pltpu.sync_copy(write.at[pl.ds(row0, RPT)],
                            out.at[pl.ds(c * N + row0, RPT)])
        plsc.subcore_barrier()


def _head_body(F_ref, Wphi_ref, bphi_ref, Wt_ref, bt_ref,
               gs_ref, phi_ref, ts_ref):
    s0 = jnp.sum(jnp.maximum(F_ref[0], 0.0), axis=0, keepdims=True)
    s1 = jnp.sum(jnp.maximum(F_ref[1], 0.0), axis=0, keepdims=True)
    gs = jnp.concatenate([s0, s1], axis=1) * (1.0 / N)
    gs_ref[...] = gs
    phi_ref[...] = (jnp.dot(gs, Wphi_ref[...],
                            preferred_element_type=jnp.float32)
                    + bphi_ref[...])
    ts_ref[...] = (jnp.dot(gs, Wt_ref[...],
                           preferred_element_type=jnp.float32)
                   + bt_ref[...])


def _head(F, Wphi, bphi, Wt, bt):
    return pl.pallas_call(
        _head_body,
        out_shape=[
            jax.ShapeDtypeStruct((1, H), jnp.float32),
            jax.ShapeDtypeStruct((1, 1), jnp.float32),
            jax.ShapeDtypeStruct((1, 256), jnp.float32),
        ],
    )(F, Wphi, bphi, Wt, bt)


def kernel(x, edge_index, edge_attr, W1, b1, W2, b2, Wzk, bzk, Wphi, bphi,
           Wt, bt):
    hs, zk = _encoder(x, W1, b1.reshape(1, H), W2, b2.reshape(1, H),
                      Wzk, bzk.reshape(1, 64))
    h0 = hs.reshape(2 * N, HALF)

    pad = EP - E
    packed = jnp.concatenate(
        [edge_index[0] | (edge_index[1] << 16),
         jnp.zeros((pad,), jnp.int32)]).reshape(NS * CPT, CHUNK)
    attr = jnp.concatenate(
        [edge_attr[:, 0], jnp.zeros((pad,), jnp.float32)]).reshape(
            NS * CPT, CHUNK)

    s3 = _mp_kernel(h0, packed, attr)
    F = s3.reshape(2, N, HALF)

    gs, phi_q, temporal_sig = _head(F, Wphi, bphi.reshape(1, 1), Wt,
                                    bt.reshape(1, 256))
    return (phi_q, temporal_sig, zk, gs)


# speedup vs baseline: 1.3326x; 1.0178x over previous
import functools

import jax
import jax.numpy as jnp
from jax import lax
from jax.experimental import pallas as pl
from jax.experimental.pallas import tpu as pltpu
from jax.experimental.pallas import tpu_sc as plsc

N = 10000
E = 320000
BD = 128
H = 128
HALF = H // 2
NUM_LAYERS = 3

NC = 2
NS = 16
CHUNK = 128
EPT = 20480
EP = EPT * NS
CPT = EPT // CHUNK
SUP = 8
LSUP = CPT // SUP
RPT = N // NS
ZB = 125
LANES = 16
NBUF = 4

_SPLAT_DN = lax.GatherDimensionNumbers(
    offset_dims=(), collapsed_slice_dims=(0,), start_index_map=(0,))


def _splat(vec, e):
    idx = jnp.full((LANES, 1), e, jnp.int32)
    return lax.gather(vec, idx, _SPLAT_DN, (1,),
                      mode=lax.GatherScatterMode.PROMISE_IN_BOUNDS)


def _encoder_body(x_ref, W1_ref, b1_ref, W2_ref, b2_ref, Wzk_ref, bzk_ref,
                  hs_ref, zk_ref):
    h1 = jnp.maximum(
        jnp.dot(x_ref[...], W1_ref[...], preferred_element_type=jnp.float32)
        + b1_ref[...], 0.0)
    h = (jnp.dot(h1, W2_ref[...], preferred_element_type=jnp.float32)
         + b2_ref[...])
    zk_ref[...] = (jnp.dot(h, Wzk_ref[...], preferred_element_type=jnp.float32)
                   + bzk_ref[...])
    hs_ref[0] = h[:, :HALF]
    hs_ref[1] = h[:, HALF:]


def _encoder(x, W1, b1, W2, b2, Wzk, bzk):
    BR = 1000
    return pl.pallas_call(
        _encoder_body,
        grid=(N // BR,),
        in_specs=[
            pl.BlockSpec((BR, BD), lambda i: (i, 0)),
            pl.BlockSpec((BD, H), lambda i: (0, 0)),
            pl.BlockSpec((1, H), lambda i: (0, 0)),
            pl.BlockSpec((H, H), lambda i: (0, 0)),
            pl.BlockSpec((1, H), lambda i: (0, 0)),
            pl.BlockSpec((H, 64), lambda i: (0, 0)),
            pl.BlockSpec((1, 64), lambda i: (0, 0)),
        ],
        out_specs=[
            pl.BlockSpec((2, BR, HALF), lambda i: (0, i, 0)),
            pl.BlockSpec((BR, 64), lambda i: (i, 0)),
        ],
        out_shape=[
            jax.ShapeDtypeStruct((2, N, HALF), jnp.float32),
            jax.ShapeDtypeStruct((N, 64), jnp.float32),
        ],
    )(x, W1, b1, W2, b2, Wzk, bzk)


_mesh = plsc.VectorSubcoreMesh(core_axis_name="c", subcore_axis_name="s",
                               num_cores=NC, num_subcores=NS)


@functools.partial(
    pl.kernel,
    out_type=jax.ShapeDtypeStruct((2 * N, HALF), jnp.float32),
    mesh=_mesh,
    compiler_params=pltpu.CompilerParams(use_tc_tiling_on_sc=False,
                                         needs_layout_passes=False),
    scratch_types=[
        pltpu.VMEM((2, SUP, CHUNK), jnp.int32),
        pltpu.VMEM((2, SUP, CHUNK), jnp.int32),
        pltpu.VMEM((2, SUP, CHUNK), jnp.float32),
        pltpu.VMEM((NBUF, CHUNK, HALF), jnp.float32),
        pltpu.VMEM((ZB, HALF), jnp.float32),
        pltpu.VMEM_SHARED((N, HALF), jnp.float32),
        pltpu.VMEM_SHARED((N, HALF), jnp.float32),
        pltpu.SemaphoreType.DMA,
        pltpu.SemaphoreType.DMA,
        pltpu.SemaphoreType.DMA,
        pltpu.SemaphoreType.DMA,
        pltpu.SemaphoreType.DMA,
        pltpu.SemaphoreType.DMA,
    ],
)
def _mp_kernel(h0, srcp, dstp, attrp, out,
               srcS, dstS, attrS, rows, zbuf, stateA, stateB,
               sg0, sg1, ss0, ss1, sl0, sl1):
    c = lax.axis_index("c")
    s = lax.axis_index("s")
    row0 = s * RPT
    ebase = s * CPT
    sg = (sg0, sg1)
    ss = (ss0, ss1)

    def sidx(cc):
        return srcS.at[lax.shift_right_logical(cc, 3) & 1, cc & 7]

    def didx(cc):
        return dstS.at[lax.shift_right_logical(cc, 3) & 1, cc & 7]

    def load_super(u, slot):
        off = ebase + u * SUP
        pltpu.async_copy(srcp.at[pl.ds(off, SUP)], srcS.at[slot], sl0)
        pltpu.async_copy(dstp.at[pl.ds(off, SUP)], dstS.at[slot], sl0)
        pltpu.async_copy(attrp.at[pl.ds(off, SUP)], attrS.at[slot], sl0)

    def wait_super(slot):
        pltpu.make_async_copy(srcp.at[pl.ds(0, SUP)], srcS.at[slot],
                              sl0).wait()
        pltpu.make_async_copy(dstp.at[pl.ds(0, SUP)], dstS.at[slot],
                              sl0).wait()
        pltpu.make_async_copy(attrp.at[pl.ds(0, SUP)], attrS.at[slot],
                              sl0).wait()

    pltpu.sync_copy(h0.at[pl.ds(c * N + row0, RPT)],
                    stateA.at[pl.ds(row0, RPT)])

    def zb_body(i, carry):
        for cs in range(HALF // LANES):
            zbuf[i, pl.ds(cs * LANES, LANES)] = jnp.zeros((LANES,),
                                                          jnp.float32)
        return carry

    lax.fori_loop(0, ZB, zb_body, 0)
    for p in range(RPT // ZB):
        pltpu.sync_copy(zbuf, stateB.at[pl.ds(row0 + p * ZB, ZB)])
    plsc.subcore_barrier()

    for layer in range(NUM_LAYERS):
        read = stateA if layer % 2 == 0 else stateB
        write = stateB if layer % 2 == 0 else stateA
        apply_relu = layer > 0

        load_super(0, 0)
        wait_super(0)
        load_super(1, 1)
        pltpu.async_copy(read.at[sidx(0)], rows.at[0], sg0)
        pltpu.async_copy(read.at[sidx(1)], rows.at[1], sg1)

        def ring_body(it4, carry):
            for b in range(NBUF):
                cc = it4 * NBUF + b
                pb = b % 2
                semg = sg[pb]
                sems = ss[pb]
                pltpu.make_async_copy(read.at[sidx(cc)],
                                      rows.at[b], semg).wait()
                @pl.when(cc >= 2)
                def _():
                    pltpu.make_async_copy(
                        rows.at[(b + 2) % NBUF],
                        write.at[didx(cc - 2)], sems).wait()

                if b == 0:
                    nu0 = lax.shift_right_logical(cc, 3) + 1

                    @pl.when(jnp.logical_and((cc & 7) == 0,
                                             jnp.logical_and(
                                                 cc >= SUP,
                                                 nu0 <= LSUP - 1)))
                    def _():
                        load_super(nu0, nu0 & 1)
                if b == 2:
                    nu2 = lax.shift_right_logical(cc, 3) + 1

                    @pl.when(jnp.logical_and((cc & 7) == SUP - 2,
                                             nu2 <= LSUP - 1))
                    def _():
                        wait_super(nu2 & 1)

                @pl.when(cc + 2 < CPT)
                def _():
                    pltpu.async_copy(read.at[sidx(cc + 2)],
                                     rows.at[(b + 2) % NBUF], semg)

                sup_slot = lax.shift_right_logical(cc, 3) & 1
                ic = cc & 7

                @plsc.parallel_loop(0, CHUNK // LANES)
                def group_body(k):
                    av = attrS[sup_slot, ic, pl.ds(k * LANES, LANES)]
                    for e in range(LANES):
                        ae = _splat(av, e)
                        j = k * LANES + e
                        for cs in range(HALF // LANES):
                            vsl = pl.ds(cs * LANES, LANES)
                            r = rows[b, j, vsl]
                            if apply_relu:
                                r = jnp.maximum(r, 0.0)
                            rows[b, j, vsl] = r * ae
                pltpu.async_copy(rows.at[b], write.at[didx(cc)], sems,
                                 add=True)
            return carry

        lax.fori_loop(0, CPT // NBUF, ring_body, 0)
        pltpu.make_async_copy(rows.at[(CPT - 2) % NBUF],
                              write.at[didx(CPT - 2)], ss[0]).wait()
        pltpu.make_async_copy(rows.at[(CPT - 1) % NBUF],
                              write.at[didx(CPT - 1)], ss[1]).wait()
        plsc.subcore_barrier()

        if layer < NUM_LAYERS - 1:
            for p in range(RPT // ZB):
                pltpu.sync_copy(zbuf, read.at[pl.ds(row0 + p * ZB, ZB)])
        else:
            pltpu.sync_copy(write.at[pl.ds(row0, RPT)],
                            out.at[pl.ds(c * N + row0, RPT)])
        plsc.subcore_barrier()


def _head_body(F_ref, Wphi_ref, bphi_ref, Wt_ref, bt_ref,
               gs_ref, phi_ref, ts_ref):
    s0 = jnp.sum(jnp.maximum(F_ref[0], 0.0), axis=0, keepdims=True)
    s1 = jnp.sum(jnp.maximum(F_ref[1], 0.0), axis=0, keepdims=True)
    gs = jnp.concatenate([s0, s1], axis=1) * (1.0 / N)
    gs_ref[...] = gs
    phi_ref[...] = (jnp.dot(gs, Wphi_ref[...],
                            preferred_element_type=jnp.float32)
                    + bphi_ref[...])
    ts_ref[...] = (jnp.dot(gs, Wt_ref[...],
                           preferred_element_type=jnp.float32)
                   + bt_ref[...])


def _head(F, Wphi, bphi, Wt, bt):
    return pl.pallas_call(
        _head_body,
        out_shape=[
            jax.ShapeDtypeStruct((1, H), jnp.float32),
            jax.ShapeDtypeStruct((1, 1), jnp.float32),
            jax.ShapeDtypeStruct((1, 256), jnp.float32),
        ],
    )(F, Wphi, bphi, Wt, bt)


def kernel(x, edge_index, edge_attr, W1, b1, W2, b2, Wzk, bzk, Wphi, bphi,
           Wt, bt):
    hs, zk = _encoder(x, W1, b1.reshape(1, H), W2, b2.reshape(1, H),
                      Wzk, bzk.reshape(1, 64))
    h0 = hs.reshape(2 * N, HALF)

    pad = EP - E
    zpad = jnp.zeros((pad,), jnp.int32)
    src_t = jnp.concatenate([edge_index[0], zpad]).reshape(NS * CPT, CHUNK)
    dst_t = jnp.concatenate([edge_index[1], zpad]).reshape(NS * CPT, CHUNK)
    attr = jnp.concatenate(
        [edge_attr[:, 0], jnp.zeros((pad,), jnp.float32)]).reshape(
            NS * CPT, CHUNK)

    s3 = _mp_kernel(h0, src_t, dst_t, attr)
    F = s3.reshape(2, N, HALF)

    gs, phi_q, temporal_sig = _head(F, Wphi, bphi.reshape(1, 1), Wt,
                                    bt.reshape(1, 256))
    return (phi_q, temporal_sig, zk, gs)
